# baseline (speedup 1.0000x reference)
import functools
import math

import jax
import jax.numpy as jnp
from jax import lax
from jax.experimental import pallas as pl
from jax.experimental.pallas import tpu as pltpu
from jax.experimental.pallas import tpu_sc as plsc

EPS = 1e-5

_SC_CORES = 2
_SC_SUBCORES = 16
_NW = _SC_CORES * _SC_SUBCORES


def _layer_norm(x, g, b):
  mu = jnp.mean(x, axis=-1, keepdims=True)
  xc = x - mu
  var = jnp.mean(xc * xc, axis=-1, keepdims=True)
  return xc * lax.rsqrt(var + EPS) * g + b


def _sc_gather(x2d, idx):
  m = idx.shape[0]
  d = x2d.shape[1]
  rows_per_w = m // _NW
  chunk = 8
  n_chunks = rows_per_w // chunk
  mesh = plsc.VectorSubcoreMesh(core_axis_name="c", subcore_axis_name="s")

  @functools.partial(
      pl.kernel,
      mesh=mesh,
      out_type=jax.ShapeDtypeStruct((m, d), jnp.float32),
      scratch_types=[
          pltpu.VMEM((chunk,), jnp.int32),
          pltpu.VMEM((chunk, d), jnp.float32),
          pltpu.SemaphoreType.DMA,
      ],
  )
  def gather_kernel(x_hbm, idx_hbm, out_hbm, idx_v, rows_v, sem):
    wid = lax.axis_index("s") * _SC_CORES + lax.axis_index("c")
    base = wid * rows_per_w
    for c in range(n_chunks):
      off = base + c * chunk
      pltpu.sync_copy(idx_hbm.at[pl.ds(off, chunk)], idx_v)
      pltpu.async_copy(x_hbm.at[idx_v], rows_v, sem).wait()
      pltpu.sync_copy(rows_v, out_hbm.at[pl.ds(off, chunk)])

  return gather_kernel(x2d, idx)


def _ln_body(x_ref, g_ref, b_ref, o_ref):
  o_ref[...] = _layer_norm(x_ref[...], g_ref[...], b_ref[...])


def _ln_all(x, g, b):
  n, l, c = x.shape
  bw = 128
  grid = (n // bw,)
  return pl.pallas_call(
      _ln_body,
      grid=grid,
      in_specs=[
          pl.BlockSpec((bw, l, c), lambda i: (i, 0, 0)),
          pl.BlockSpec((1, 1, c), lambda i: (0, 0, 0)),
          pl.BlockSpec((1, 1, c), lambda i: (0, 0, 0)),
      ],
      out_specs=pl.BlockSpec((bw, l, c), lambda i: (i, 0, 0)),
      out_shape=jax.ShapeDtypeStruct((n, l, c), jnp.float32),
      compiler_params=pltpu.CompilerParams(
          dimension_semantics=("arbitrary",),
      ),
  )(x, g.reshape(1, 1, c), b.reshape(1, 1, c))


def _block_body(idx_sref, xs_ref, xin_ref, g1_ref, b1_ref, wqkv_ref, bqkv_ref,
                wp_ref, bp_ref, g2_ref, b2_ref, w1_ref, bf1_ref, w2_ref,
                bf2_ref, out_ref, y_scratch, sem, *, G, L, C):
  i = pl.program_id(0)
  dim_head = 32
  num_heads = C // dim_head
  scale = dim_head ** (-0.5)

  xs = xs_ref[...].reshape(G * L, C)
  h = _layer_norm(xs, g1_ref[...], b1_ref[...])
  shortcut = h
  qkv = jnp.dot(h, wqkv_ref[...], preferred_element_type=jnp.float32)
  qkv = qkv + bqkv_ref[...]

  outs = []
  for w in range(G):
    a = qkv[w * L:(w + 1) * L]
    heads = []
    for hd in range(num_heads):
      q = a[:, hd * 3 * dim_head:hd * 3 * dim_head + dim_head]
      k = a[:, hd * 3 * dim_head + dim_head:hd * 3 * dim_head + 2 * dim_head]
      v = a[:, hd * 3 * dim_head + 2 * dim_head:hd * 3 * dim_head + 3 * dim_head]
      s = lax.dot_general(q, k, (((1,), (1,)), ((), ())),
                          preferred_element_type=jnp.float32) * scale
      s = s - jnp.max(s, axis=-1, keepdims=True)
      e = jnp.exp(s)
      p = e / jnp.sum(e, axis=-1, keepdims=True)
      heads.append(jnp.dot(p, v, preferred_element_type=jnp.float32))
    outs.append(jnp.concatenate(heads, axis=1))
  attn = jnp.concatenate(outs, axis=0)

  y = jnp.dot(attn, wp_ref[...], preferred_element_type=jnp.float32)
  y = y + bp_ref[...] + shortcut
  h2 = _layer_norm(y, g2_ref[...], b2_ref[...])
  f = jnp.dot(h2, w1_ref[...], preferred_element_type=jnp.float32)
  f = f + bf1_ref[...]
  f = 0.5 * f * (1.0 + lax.erf(f * (1.0 / math.sqrt(2.0))))
  y = y + jnp.dot(f, w2_ref[...], preferred_element_type=jnp.float32)
  y = y + bf2_ref[...]
  y_scratch[...] = y.reshape(G, L, C)

  copies = []
  for w in range(G):
    widx = idx_sref[i * G + w]
    copies.append(
        pltpu.make_async_copy(y_scratch.at[w], out_ref.at[widx], sem))
  for cp in copies:
    cp.start()
  for cp in copies:
    cp.wait()


def _block_scatter(idx, xs, xln, g1, b1, wqkv, bqkv, wp, bp, g2, b2, w1, bf1,
                   w2, bf2):
  m, l, c = xs.shape
  n = xln.shape[0]
  g = 8
  grid = (m // g,)
  dff = w1.shape[1]
  body = functools.partial(_block_body, G=g, L=l, C=c)
  grid_spec = pltpu.PrefetchScalarGridSpec(
      num_scalar_prefetch=1,
      grid=grid,
      in_specs=[
          pl.BlockSpec((g, l, c), lambda i, idx_s: (i, 0, 0)),
          pl.BlockSpec(memory_space=pl.ANY),
          pl.BlockSpec((1, c), lambda i, idx_s: (0, 0)),
          pl.BlockSpec((1, c), lambda i, idx_s: (0, 0)),
          pl.BlockSpec((c, 3 * c), lambda i, idx_s: (0, 0)),
          pl.BlockSpec((1, 3 * c), lambda i, idx_s: (0, 0)),
          pl.BlockSpec((c, c), lambda i, idx_s: (0, 0)),
          pl.BlockSpec((1, c), lambda i, idx_s: (0, 0)),
          pl.BlockSpec((1, c), lambda i, idx_s: (0, 0)),
          pl.BlockSpec((1, c), lambda i, idx_s: (0, 0)),
          pl.BlockSpec((c, dff), lambda i, idx_s: (0, 0)),
          pl.BlockSpec((1, dff), lambda i, idx_s: (0, 0)),
          pl.BlockSpec((dff, c), lambda i, idx_s: (0, 0)),
          pl.BlockSpec((1, c), lambda i, idx_s: (0, 0)),
      ],
      out_specs=pl.BlockSpec(memory_space=pl.ANY),
      scratch_shapes=[
          pltpu.VMEM((g, l, c), jnp.float32),
          pltpu.SemaphoreType.DMA,
      ],
  )
  return pl.pallas_call(
      body,
      grid_spec=grid_spec,
      out_shape=jax.ShapeDtypeStruct((n, l, c), jnp.float32),
      input_output_aliases={2: 0},
      compiler_params=pltpu.CompilerParams(
          dimension_semantics=("arbitrary",),
      ),
  )(idx, xs, xln, g1.reshape(1, c), b1.reshape(1, c), wqkv,
    bqkv.reshape(1, 3 * c), wp, bp.reshape(1, c), g2.reshape(1, c),
    b2.reshape(1, c), w1, bf1.reshape(1, dff), w2, bf2.reshape(1, c))


def kernel(x, index, M, g1, b1, Wqkv, bqkv, Wp, bp, g2, b2, W1, bf1, W2, bf2):
  n, l, c = x.shape
  idx = index.astype(jnp.int32)
  xs = _sc_gather(x.reshape(n, l * c), idx).reshape(-1, l, c)
  xln = _ln_all(x, g1, b1)
  return _block_scatter(idx, xs, xln, g1, b1, Wqkv, bqkv, Wp, bp, g2, b2,
                        W1, bf1, W2, bf2)
